# baseline (speedup 1.0000x reference)
import jax
import jax.numpy as jnp
from jax import lax
from jax.experimental import pallas as pl
from jax.experimental.pallas import tpu as pltpu
from jax.experimental.pallas import tpu_sc as plsc

N = 10000
E = 320000
F = 128
C = 128
NG = 16
NC = 2
NS = 16
NW = NC * NS
N_PAD = 10240
EPW = E // NW
KD = 128
NCHD = 78
KA = 96
NCHA = 104
TB = 9984
TAIL = 16
RPT = N_PAD // NS
BLK = 2048


def _mesh():
  return plsc.VectorSubcoreMesh(core_axis_name="c", subcore_axis_name="s")


def _sc_deg_body(ei_hbm, zeros_hbm, ones_hbm, out_hbm,
                 deg_sh, idx_v, ones_v, sem):
  c = lax.axis_index("c")
  s = lax.axis_index("s")
  wid = c * NS + s
  pltpu.sync_copy(zeros_hbm.at[pl.ds(s * RPT, RPT)],
                  deg_sh.at[pl.ds(s * RPT, RPT)])
  pltpu.sync_copy(ones_hbm, ones_v)
  pltpu.sync_copy(ei_hbm.at[0, wid], idx_v)
  plsc.subcore_barrier()

  def fire(i, carry):
    pltpu.async_copy(ones_v, deg_sh.at[idx_v.at[pl.ds(i * KD, KD)]], sem,
                     add=True)
    return carry

  lax.fori_loop(0, NCHD, fire, 0)
  pltpu.async_copy(ones_v.at[pl.ds(0, TAIL)],
                   deg_sh.at[idx_v.at[pl.ds(TB, TAIL)]], sem, add=True)

  def drain(i, carry):
    pltpu.make_async_copy(ones_v, deg_sh.at[idx_v.at[pl.ds(0, KD)]],
                          sem).wait()
    return carry

  lax.fori_loop(0, NCHD, drain, 0)
  pltpu.make_async_copy(ones_v.at[pl.ds(0, TAIL)],
                        deg_sh.at[idx_v.at[pl.ds(TB, TAIL)]], sem).wait()
  plsc.subcore_barrier()
  pltpu.sync_copy(deg_sh.at[pl.ds(s * RPT, RPT)],
                  out_hbm.at[c, pl.ds(s * RPT, RPT)])


_sc_deg = pl.kernel(
    _sc_deg_body,
    out_type=jax.ShapeDtypeStruct((NC, N_PAD), jnp.float32),
    mesh=_mesh(),
    scratch_types=[
        pltpu.VMEM_SHARED((N_PAD,), jnp.float32),
        pltpu.VMEM((EPW,), jnp.int32),
        pltpu.VMEM((KD,), jnp.float32),
        pltpu.SemaphoreType.DMA,
    ],
)


def _sc_agg_body(ei_hbm, table_hbm, zrows_hbm, out_hbm,
                 acc_sh, sidx_v, didx_v, rows0, rows1, rows_t, sem0, sem1):
  c = lax.axis_index("c")
  s = lax.axis_index("s")
  wid = c * NS + s
  pltpu.sync_copy(zrows_hbm.at[pl.ds(s * RPT, RPT)],
                  acc_sh.at[pl.ds(s * RPT, RPT)])
  pltpu.sync_copy(ei_hbm.at[0, wid], sidx_v)
  pltpu.sync_copy(ei_hbm.at[1, wid], didx_v)
  plsc.subcore_barrier()

  rows = (rows0, rows1)
  sems = (sem0, sem1)
  for b in range(2):
    pltpu.async_copy(table_hbm.at[sidx_v.at[pl.ds(b * KA, KA)]],
                     rows[b], sems[b])

  def chunk2(g, carry):
    for b in range(2):
      i = 2 * g + b
      pltpu.make_async_copy(table_hbm.at[sidx_v.at[pl.ds(i * KA, KA)]],
                            rows[b], sems[b]).wait()
      pltpu.sync_copy(rows[b], acc_sh.at[didx_v.at[pl.ds(i * KA, KA)]],
                      add=True)

      @pl.when(i + 2 < NCHA)
      def _():
        pltpu.async_copy(
            table_hbm.at[sidx_v.at[pl.ds((i + 2) * KA, KA)]],
            rows[b], sems[b])

    return carry

  lax.fori_loop(0, NCHA // 2, chunk2, 0)
  pltpu.async_copy(table_hbm.at[sidx_v.at[pl.ds(TB, TAIL)]],
                   rows_t, sem0).wait()
  pltpu.sync_copy(rows_t, acc_sh.at[didx_v.at[pl.ds(TB, TAIL)]], add=True)
  plsc.subcore_barrier()
  pltpu.sync_copy(acc_sh.at[pl.ds(s * RPT, RPT)],
                  out_hbm.at[c, pl.ds(s * RPT, RPT)])


def _make_sc_agg(dtype):
  return pl.kernel(
      _sc_agg_body,
      out_type=jax.ShapeDtypeStruct((NC, N_PAD, C), dtype),
      mesh=_mesh(),
      scratch_types=[
          pltpu.VMEM_SHARED((N_PAD, C), dtype),
          pltpu.VMEM((EPW,), jnp.int32),
          pltpu.VMEM((EPW,), jnp.int32),
          pltpu.VMEM((KA, C), dtype),
          pltpu.VMEM((KA, C), dtype),
          pltpu.VMEM((TAIL, C), dtype),
          pltpu.SemaphoreType.DMA,
          pltpu.SemaphoreType.DMA,
      ],
  )


_sc_agg = _make_sc_agg(jnp.float32)


def _tc1_body(d2, x, w0, w1, b, p1, g1s, dinv_o):
  deg = d2[0, 0] + d2[1, 0]
  dinv1 = jnp.where(deg > 0, lax.rsqrt(jnp.maximum(deg, 1e-12)), 0.0)
  dinv = dinv1.reshape(BLK, 1)
  xb = x[...]
  p1[...] = jnp.dot(xb, w0[...], preferred_element_type=jnp.float32) + b[...]
  g1s[...] = (dinv * jnp.dot(xb, w1[...],
                             preferred_element_type=jnp.float32)
              ).astype(g1s.dtype)
  dinv_o[...] = dinv1


_tc1 = pl.pallas_call(
    _tc1_body,
    grid=(N_PAD // BLK,),
    in_specs=[
        pl.BlockSpec((NC, 1, BLK), lambda i: (0, 0, i)),
        pl.BlockSpec((BLK, F), lambda i: (i, 0)),
        pl.BlockSpec((F, C), lambda i: (0, 0)),
        pl.BlockSpec((F, C), lambda i: (0, 0)),
        pl.BlockSpec((1, C), lambda i: (0, 0)),
    ],
    out_specs=[
        pl.BlockSpec((BLK, C), lambda i: (i, 0)),
        pl.BlockSpec((BLK, C), lambda i: (i, 0)),
        pl.BlockSpec((BLK,), lambda i: (i,)),
    ],
    out_shape=[
        jax.ShapeDtypeStruct((N_PAD, C), jnp.float32),
        jax.ShapeDtypeStruct((N_PAD, C), jnp.float32),
        jax.ShapeDtypeStruct((N_PAD,), jnp.float32),
    ],
)


def _tc2_body(p1, a2, dinv_r, w0, w1, b, p2, g2s):
  dinv = dinv_r[...].reshape(BLK, 1)
  acc = a2[0].astype(jnp.float32) + a2[1].astype(jnp.float32)
  h1 = jnp.maximum(p1[...] - dinv * acc, 0.0)
  p2[...] = jnp.dot(h1, w0[...], preferred_element_type=jnp.float32) + b[...]
  g2s[...] = (dinv * jnp.dot(h1, w1[...],
                             preferred_element_type=jnp.float32)
              ).astype(g2s.dtype)


_tc2 = pl.pallas_call(
    _tc2_body,
    grid=(N_PAD // BLK,),
    in_specs=[
        pl.BlockSpec((BLK, C), lambda i: (i, 0)),
        pl.BlockSpec((NC, BLK, C), lambda i: (0, i, 0)),
        pl.BlockSpec((BLK,), lambda i: (i,)),
        pl.BlockSpec((C, C), lambda i: (0, 0)),
        pl.BlockSpec((C, C), lambda i: (0, 0)),
        pl.BlockSpec((1, C), lambda i: (0, 0)),
    ],
    out_specs=[
        pl.BlockSpec((BLK, C), lambda i: (i, 0)),
        pl.BlockSpec((BLK, C), lambda i: (i, 0)),
    ],
    out_shape=[
        jax.ShapeDtypeStruct((N_PAD, C), jnp.float32),
        jax.ShapeDtypeStruct((N_PAD, C), jnp.float32),
    ],
)


def _tc3_body(p2, a2, dinv_r, rw, rb, score):
  dinv = dinv_r[...].reshape(BLK, 1)
  acc = a2[0].astype(jnp.float32) + a2[1].astype(jnp.float32)
  h2 = jnp.maximum(p2[...] - dinv * acc, 0.0)
  score[...] = jnp.sum(h2 * rw[...], axis=1, keepdims=True) + rb[...]


_tc3 = pl.pallas_call(
    _tc3_body,
    grid=(N_PAD // BLK,),
    in_specs=[
        pl.BlockSpec((BLK, C), lambda i: (i, 0)),
        pl.BlockSpec((NC, BLK, C), lambda i: (0, i, 0)),
        pl.BlockSpec((BLK,), lambda i: (i,)),
        pl.BlockSpec((1, C), lambda i: (0, 0)),
        pl.BlockSpec((1, 1), lambda i: (0, 0)),
    ],
    out_specs=[pl.BlockSpec((BLK, 1), lambda i: (i, 0))],
    out_shape=[jax.ShapeDtypeStruct((N_PAD, 1), jnp.float32)],
)


def _tc4_body(s_ref, b_ref, y_ref):
  s = s_ref[...]
  bt = b_ref[...]
  msel = jnp.zeros_like(s)
  for g in range(NG):
    mask = bt == g
    mg = jnp.max(jnp.where(mask, s, -jnp.inf))
    mg = jnp.where(mg == -jnp.inf, 0.0, mg)
    msel = jnp.where(mask, mg, msel)
  e = jnp.exp(s - msel)
  zsel = jnp.ones_like(s)
  for g in range(NG):
    mask = bt == g
    zg = jnp.sum(jnp.where(mask, e, 0.0))
    zsel = jnp.where(mask, zg, zsel)
  y_ref[...] = e / zsel


_tc4 = pl.pallas_call(
    _tc4_body,
    out_shape=jax.ShapeDtypeStruct((N_PAD // 128, 128), jnp.float32),
)


@jax.jit
def kernel(x, edge_index, batch, c1_w0, c1_w1, c1_b, c2_w0, c2_w1, c2_b,
           r_w, r_b):
  ei_p = edge_index.reshape(2, NW, EPW)
  x_p = jnp.pad(x, ((0, N_PAD - N), (0, 0)))
  zeros1 = jnp.zeros((N_PAD,), jnp.float32)
  ones1 = jnp.ones((KD,), jnp.float32)
  zrows = jnp.zeros((N_PAD, C), jnp.float32)

  deg2 = _sc_deg(ei_p, zeros1, ones1)
  p1, g1s, dinv = _tc1(deg2.reshape(NC, 1, N_PAD), x_p, c1_w0, c1_w1,
                       c1_b.reshape(1, C))
  acc1 = _sc_agg(ei_p, g1s, zrows)
  p2, g2s = _tc2(p1, acc1, dinv, c2_w0, c2_w1, c2_b.reshape(1, C))
  acc2 = _sc_agg(ei_p, g2s, zrows)
  (score,) = _tc3(p2, acc2, dinv, r_w.reshape(1, C), r_b.reshape(1, 1))
  batch_p = jnp.pad(batch, (0, N_PAD - N), constant_values=NG)
  y2 = _tc4(score.reshape(N_PAD // 128, 128),
            batch_p.reshape(N_PAD // 128, 128))
  return y2.reshape(-1)[:N]


# speedup vs baseline: 1.0129x; 1.0129x over previous
import jax
import jax.numpy as jnp
from jax import lax
from jax.experimental import pallas as pl
from jax.experimental.pallas import tpu as pltpu
from jax.experimental.pallas import tpu_sc as plsc

N = 10000
E = 320000
F = 128
C = 128
NG = 16
NC = 2
NS = 16
NW = NC * NS
N_PAD = 10240
EPW = E // NW
KD = 128
NCHD = 78
KA = 128
HEP = 4992
NCHH = 39
TB = 9984
TAIL = 16
RPT = N_PAD // NS
BLK = 2048


def _mesh():
  return plsc.VectorSubcoreMesh(core_axis_name="c", subcore_axis_name="s")


def _sc_deg_body(ei_hbm, zeros_hbm, ones_hbm, out_hbm,
                 deg_sh, idx_v, ones_v, sem):
  c = lax.axis_index("c")
  s = lax.axis_index("s")
  wid = c * NS + s
  pltpu.sync_copy(zeros_hbm.at[pl.ds(s * RPT, RPT)],
                  deg_sh.at[pl.ds(s * RPT, RPT)])
  pltpu.sync_copy(ones_hbm, ones_v)
  pltpu.sync_copy(ei_hbm.at[0, wid], idx_v)
  plsc.subcore_barrier()

  def fire(i, carry):
    pltpu.async_copy(ones_v, deg_sh.at[idx_v.at[pl.ds(i * KD, KD)]], sem,
                     add=True)
    return carry

  lax.fori_loop(0, NCHD, fire, 0)
  pltpu.async_copy(ones_v.at[pl.ds(0, TAIL)],
                   deg_sh.at[idx_v.at[pl.ds(TB, TAIL)]], sem, add=True)

  def drain(i, carry):
    pltpu.make_async_copy(ones_v, deg_sh.at[idx_v.at[pl.ds(0, KD)]],
                          sem).wait()
    return carry

  lax.fori_loop(0, NCHD, drain, 0)
  pltpu.make_async_copy(ones_v.at[pl.ds(0, TAIL)],
                        deg_sh.at[idx_v.at[pl.ds(TB, TAIL)]], sem).wait()
  plsc.subcore_barrier()
  pltpu.sync_copy(deg_sh.at[pl.ds(s * RPT, RPT)],
                  out_hbm.at[c, pl.ds(s * RPT, RPT)])


_sc_deg = pl.kernel(
    _sc_deg_body,
    out_type=jax.ShapeDtypeStruct((NC, N_PAD), jnp.float32),
    mesh=_mesh(),
    scratch_types=[
        pltpu.VMEM_SHARED((N_PAD,), jnp.float32),
        pltpu.VMEM((EPW,), jnp.int32),
        pltpu.VMEM((KD,), jnp.float32),
        pltpu.SemaphoreType.DMA,
    ],
)


def _sc_agg_body(ei_hbm, table_hbm, zrows_hbm, out_hbm,
                 acc_sh, sidx_v, didx_v, rows0, rows1, rows_t,
                 stail_v, dtail_v, sem0, sem1):
  c = lax.axis_index("c")
  s = lax.axis_index("s")
  wid = c * NS + s
  pltpu.sync_copy(zrows_hbm.at[pl.ds(s * RPT, RPT)],
                  acc_sh.at[pl.ds(s * RPT, RPT)])
  plsc.subcore_barrier()

  rows = (rows0, rows1)
  sems = (sem0, sem1)
  for h in range(2):
    pltpu.sync_copy(ei_hbm.at[0, wid, pl.ds(h * HEP, HEP)], sidx_v)
    pltpu.sync_copy(ei_hbm.at[1, wid, pl.ds(h * HEP, HEP)], didx_v)
    for b in range(2):
      pltpu.async_copy(table_hbm.at[sidx_v.at[pl.ds(b * KA, KA)]],
                       rows[b], sems[b])

    def chunk2(g, carry):
      for b in range(2):
        i = 2 * g + b
        pltpu.make_async_copy(table_hbm.at[sidx_v.at[pl.ds(i * KA, KA)]],
                              rows[b], sems[b]).wait()
        pltpu.sync_copy(rows[b], acc_sh.at[didx_v.at[pl.ds(i * KA, KA)]],
                        add=True)

        @pl.when(i + 2 < NCHH)
        def _():
          pltpu.async_copy(
              table_hbm.at[sidx_v.at[pl.ds((i + 2) * KA, KA)]],
              rows[b], sems[b])

      return carry

    lax.fori_loop(0, NCHH // 2, chunk2, 0)
    i_last = NCHH - 1
    pltpu.make_async_copy(table_hbm.at[sidx_v.at[pl.ds(i_last * KA, KA)]],
                          rows[0], sems[0]).wait()
    pltpu.sync_copy(rows[0], acc_sh.at[didx_v.at[pl.ds(i_last * KA, KA)]],
                    add=True)
  pltpu.sync_copy(ei_hbm.at[0, wid, pl.ds(TB, TAIL)], stail_v)
  pltpu.sync_copy(ei_hbm.at[1, wid, pl.ds(TB, TAIL)], dtail_v)
  pltpu.async_copy(table_hbm.at[stail_v], rows_t, sem0).wait()
  pltpu.sync_copy(rows_t, acc_sh.at[dtail_v], add=True)
  plsc.subcore_barrier()
  pltpu.sync_copy(acc_sh.at[pl.ds(s * RPT, RPT)],
                  out_hbm.at[c, pl.ds(s * RPT, RPT)])


def _make_sc_agg(dtype):
  return pl.kernel(
      _sc_agg_body,
      out_type=jax.ShapeDtypeStruct((NC, N_PAD, C), dtype),
      mesh=_mesh(),
      scratch_types=[
          pltpu.VMEM_SHARED((N_PAD, C), dtype),
          pltpu.VMEM((HEP,), jnp.int32),
          pltpu.VMEM((HEP,), jnp.int32),
          pltpu.VMEM((KA, C), dtype),
          pltpu.VMEM((KA, C), dtype),
          pltpu.VMEM((TAIL, C), dtype),
          pltpu.VMEM((TAIL,), jnp.int32),
          pltpu.VMEM((TAIL,), jnp.int32),
          pltpu.SemaphoreType.DMA,
          pltpu.SemaphoreType.DMA,
      ],
  )


_sc_agg = _make_sc_agg(jnp.float32)


def _tc1_body(d2, x, w0, w1, b, p1, g1s, dinv_o):
  deg = d2[0, 0] + d2[1, 0]
  dinv1 = jnp.where(deg > 0, lax.rsqrt(jnp.maximum(deg, 1e-12)), 0.0)
  dinv = dinv1.reshape(BLK, 1)
  xb = x[...]
  p1[...] = jnp.dot(xb, w0[...], preferred_element_type=jnp.float32) + b[...]
  g1s[...] = (dinv * jnp.dot(xb, w1[...],
                             preferred_element_type=jnp.float32)
              ).astype(g1s.dtype)
  dinv_o[...] = dinv1


_tc1 = pl.pallas_call(
    _tc1_body,
    grid=(N_PAD // BLK,),
    in_specs=[
        pl.BlockSpec((NC, 1, BLK), lambda i: (0, 0, i)),
        pl.BlockSpec((BLK, F), lambda i: (i, 0)),
        pl.BlockSpec((F, C), lambda i: (0, 0)),
        pl.BlockSpec((F, C), lambda i: (0, 0)),
        pl.BlockSpec((1, C), lambda i: (0, 0)),
    ],
    out_specs=[
        pl.BlockSpec((BLK, C), lambda i: (i, 0)),
        pl.BlockSpec((BLK, C), lambda i: (i, 0)),
        pl.BlockSpec((BLK,), lambda i: (i,)),
    ],
    out_shape=[
        jax.ShapeDtypeStruct((N_PAD, C), jnp.float32),
        jax.ShapeDtypeStruct((N_PAD, C), jnp.float32),
        jax.ShapeDtypeStruct((N_PAD,), jnp.float32),
    ],
)


def _tc2_body(p1, a2, dinv_r, w0, w1, b, p2, g2s):
  dinv = dinv_r[...].reshape(BLK, 1)
  acc = a2[0].astype(jnp.float32) + a2[1].astype(jnp.float32)
  h1 = jnp.maximum(p1[...] - dinv * acc, 0.0)
  p2[...] = jnp.dot(h1, w0[...], preferred_element_type=jnp.float32) + b[...]
  g2s[...] = (dinv * jnp.dot(h1, w1[...],
                             preferred_element_type=jnp.float32)
              ).astype(g2s.dtype)


_tc2 = pl.pallas_call(
    _tc2_body,
    grid=(N_PAD // BLK,),
    in_specs=[
        pl.BlockSpec((BLK, C), lambda i: (i, 0)),
        pl.BlockSpec((NC, BLK, C), lambda i: (0, i, 0)),
        pl.BlockSpec((BLK,), lambda i: (i,)),
        pl.BlockSpec((C, C), lambda i: (0, 0)),
        pl.BlockSpec((C, C), lambda i: (0, 0)),
        pl.BlockSpec((1, C), lambda i: (0, 0)),
    ],
    out_specs=[
        pl.BlockSpec((BLK, C), lambda i: (i, 0)),
        pl.BlockSpec((BLK, C), lambda i: (i, 0)),
    ],
    out_shape=[
        jax.ShapeDtypeStruct((N_PAD, C), jnp.float32),
        jax.ShapeDtypeStruct((N_PAD, C), jnp.float32),
    ],
)


def _tc3_body(p2, a2, dinv_r, rw, rb, score):
  dinv = dinv_r[...].reshape(BLK, 1)
  acc = a2[0].astype(jnp.float32) + a2[1].astype(jnp.float32)
  h2 = jnp.maximum(p2[...] - dinv * acc, 0.0)
  score[...] = jnp.sum(h2 * rw[...], axis=1, keepdims=True) + rb[...]


_tc3 = pl.pallas_call(
    _tc3_body,
    grid=(N_PAD // BLK,),
    in_specs=[
        pl.BlockSpec((BLK, C), lambda i: (i, 0)),
        pl.BlockSpec((NC, BLK, C), lambda i: (0, i, 0)),
        pl.BlockSpec((BLK,), lambda i: (i,)),
        pl.BlockSpec((1, C), lambda i: (0, 0)),
        pl.BlockSpec((1, 1), lambda i: (0, 0)),
    ],
    out_specs=[pl.BlockSpec((BLK, 1), lambda i: (i, 0))],
    out_shape=[jax.ShapeDtypeStruct((N_PAD, 1), jnp.float32)],
)


def _tc4_body(s_ref, b_ref, y_ref):
  s = s_ref[...]
  bt = b_ref[...]
  msel = jnp.zeros_like(s)
  for g in range(NG):
    mask = bt == g
    mg = jnp.max(jnp.where(mask, s, -jnp.inf))
    mg = jnp.where(mg == -jnp.inf, 0.0, mg)
    msel = jnp.where(mask, mg, msel)
  e = jnp.exp(s - msel)
  zsel = jnp.ones_like(s)
  for g in range(NG):
    mask = bt == g
    zg = jnp.sum(jnp.where(mask, e, 0.0))
    zsel = jnp.where(mask, zg, zsel)
  y_ref[...] = e / zsel


_tc4 = pl.pallas_call(
    _tc4_body,
    out_shape=jax.ShapeDtypeStruct((N_PAD // 128, 128), jnp.float32),
)


@jax.jit
def kernel(x, edge_index, batch, c1_w0, c1_w1, c1_b, c2_w0, c2_w1, c2_b,
           r_w, r_b):
  ei_p = edge_index.reshape(2, NW, EPW)
  x_p = jnp.pad(x, ((0, N_PAD - N), (0, 0)))
  zeros1 = jnp.zeros((N_PAD,), jnp.float32)
  ones1 = jnp.ones((KD,), jnp.float32)
  zrows = jnp.zeros((N_PAD, C), jnp.float32)

  deg2 = _sc_deg(ei_p, zeros1, ones1)
  p1, g1s, dinv = _tc1(deg2.reshape(NC, 1, N_PAD), x_p, c1_w0, c1_w1,
                       c1_b.reshape(1, C))
  acc1 = _sc_agg(ei_p, g1s, zrows)
  p2, g2s = _tc2(p1, acc1, dinv, c2_w0, c2_w1, c2_b.reshape(1, C))
  acc2 = _sc_agg(ei_p, g2s, zrows)
  (score,) = _tc3(p2, acc2, dinv, r_w.reshape(1, C), r_b.reshape(1, 1))
  batch_p = jnp.pad(batch, (0, N_PAD - N), constant_values=NG)
  y2 = _tc4(score.reshape(N_PAD // 128, 128),
            batch_p.reshape(N_PAD // 128, 128))
  return y2.reshape(-1)[:N]
